# baseline (speedup 1.0000x reference)
import functools

import jax
import jax.numpy as jnp
from jax import lax
from jax.experimental import pallas as pl
from jax.experimental.pallas import tpu as pltpu
from jax.experimental.pallas import tpu_sc as plsc

_B, _N, _C, _K = 4, 10000, 128, 16
_DW = 80
_NC, _NS = 2, 16
_NP = 10240
_RP = _B * _NP
_CH = 8
_PER_T = _NP // _NS
_NCHUNK = _PER_T // _CH
_TROWS = _N // _NS
_HIMASK = -65536


@functools.cache
def _make_sc_gather_mean():
    mesh = plsc.VectorSubcoreMesh(
        core_axis_name="c", subcore_axis_name="s",
        num_cores=_NC, num_subcores=_NS)

    @functools.partial(
        pl.kernel,
        out_type=(
            jax.ShapeDtypeStruct((_RP, _C), jnp.float32),
            jax.ShapeDtypeStruct((_RP, 32), jnp.float32),
        ),
        mesh=mesh,
        compiler_params=pltpu.CompilerParams(
            use_tc_tiling_on_sc=False, needs_layout_passes=False),
        scratch_types=[
            pltpu.VMEM((_PER_T * _K // 128, 128), jnp.int32),
            pltpu.VMEM((4, _CH * _K, _DW), jnp.int32),
            pltpu.VMEM((2, _CH, _C), jnp.float32),
            pltpu.VMEM((2, _CH, 32), jnp.float32),
            pltpu.VMEM_SHARED((_N, _DW), jnp.int32),
            pltpu.SemaphoreType.DMA((4,)),
            pltpu.SemaphoreType.DMA((2,)),
            pltpu.SemaphoreType.DMA((2,)),
            pltpu.SemaphoreType.DMA,
        ],
    )
    def _sc_gather_mean(tab_hbm, idxg_hbm, outx_hbm, outm_hbm,
                        idx_v, rows_v, accx_v, accm_v, tab_sh,
                        rsem, sxsem, smsem, tsem):
        c = lax.axis_index("c")
        s = lax.axis_index("s")

        def _gather(j, sl):
            pltpu.async_copy(
                tab_sh.at[idx_v.at[j]],
                rows_v.at[sl], rsem.at[sl])

        def _gather_wait(sl):
            pltpu.make_async_copy(
                tab_sh.at[idx_v.at[0]],
                rows_v.at[sl], rsem.at[sl]).wait()

        inv_k = 1.0 / _K

        cbase = jnp.minimum(s * _PER_T, _N - _PER_T)

        for phase in range(2):
            b = c * 2 + phase
            pltpu.async_copy(
                tab_hbm.at[pl.ds(b * _N + s * _TROWS, _TROWS)],
                tab_sh.at[pl.ds(s * _TROWS, _TROWS)], tsem).wait()
            plsc.subcore_barrier()

            pltpu.sync_copy(
                idxg_hbm.at[pl.ds((b * _N + cbase) * _K // 128, _NCHUNK)],
                idx_v)
            base = b * _NP + cbase

            def _store(j, sl, base=base):
                row0 = base + j * _CH
                pltpu.async_copy(accx_v.at[sl],
                                 outx_hbm.at[pl.ds(row0, _CH)], sxsem.at[sl])
                pltpu.async_copy(accm_v.at[sl],
                                 outm_hbm.at[pl.ds(row0, _CH)], smsem.at[sl])

            def _store_wait(sl):
                pltpu.make_async_copy(
                    accx_v.at[sl], outx_hbm.at[pl.ds(0, _CH)],
                    sxsem.at[sl]).wait()
                pltpu.make_async_copy(
                    accm_v.at[sl], outm_hbm.at[pl.ds(0, _CH)],
                    smsem.at[sl]).wait()

            _gather(0, 0)
            _gather(1, 1)
            _gather(2, 2)

            @pl.loop(0, _NCHUNK, step=4)
            def _quad(j0):
                for sl in (0, 1, 2, 3):
                    j = j0 + sl
                    st = sl & 1

                    @pl.when(j + 3 < _NCHUNK)
                    def _():
                        _gather(j + 3, (sl + 3) & 3)

                    _gather_wait(sl)

                    @pl.when(j >= 2)
                    def _():
                        _store_wait(st)

                    @pl.loop(0, _CH)
                    def _center(cc):
                        r0 = cc * _K
                        for d in range(_DW // 16):
                            w0 = rows_v[sl, r0, pl.ds(d * 16, 16)]
                            alo = plsc.bitcast(w0 << 16, jnp.float32)
                            ahi = plsc.bitcast(w0, jnp.float32)
                            for k in range(1, _K):
                                wk = rows_v[sl, r0 + k, pl.ds(d * 16, 16)]
                                alo = alo + plsc.bitcast(wk << 16, jnp.float32)
                                ahi = ahi + plsc.bitcast(wk, jnp.float32)
                            alo = alo * inv_k
                            ahi = ahi * inv_k
                            if d < 4:
                                accx_v[st, cc, pl.ds(d * 16, 16)] = alo
                                accx_v[st, cc, pl.ds(64 + d * 16, 16)] = ahi
                            else:
                                accm_v[st, cc, pl.ds(0, 16)] = alo
                                accm_v[st, cc, pl.ds(16, 16)] = ahi

                    _store(j, st)

            _store_wait(0)
            _store_wait(1)
            plsc.subcore_barrier()

    return _sc_gather_mean


def _pack_body(x_ref, pos_ref, idx_ref, tab_ref, idx2_ref, pq_ref):
    him = jnp.uint32(0xffff0000)
    xb = lax.bitcast_convert_type(x_ref[...], jnp.uint32)
    wx = (xb[:, 64:128] & him) | (xb[:, 0:64] >> 16)
    p3 = pos_ref[...]
    px, py, pz = p3[:, 0:1], p3[:, 1:2], p3[:, 2:3]
    q = px * px + py * py + pz * pz
    pb = lax.bitcast_convert_type(
        jnp.concatenate([px, py, pz, q], axis=1), jnp.uint32)
    w64 = (pb[:, 1:2] & him) | (pb[:, 0:1] >> 16)
    w65 = (pb[:, 3:4] & him) | (pb[:, 2:3] >> 16)
    zpad = jnp.zeros((wx.shape[0], _DW - 66), jnp.uint32)
    tab_ref[...] = lax.bitcast_convert_type(
        jnp.concatenate([wx, w64, w65, zpad], axis=1), jnp.int32)
    pq_ref[...] = jnp.concatenate(
        [px, py, pz, q, jnp.zeros((wx.shape[0], 4), jnp.float32)], axis=1)
    ir3 = idx_ref[...].reshape(_BP // 8, 8, _K)
    for t in range(8):
        idx2_ref[:, pl.ds(_K * t, _K)] = ir3[:, t, :]


_BP = 1600

_tc_pack = pl.pallas_call(
    _pack_body,
    grid=(_B * _N // _BP,),
    in_specs=[
        pl.BlockSpec((_BP, _C), lambda i: (i, 0)),
        pl.BlockSpec((_BP, 3), lambda i: (i, 0)),
        pl.BlockSpec((_BP, _K), lambda i: (i, 0)),
    ],
    out_specs=[
        pl.BlockSpec((_BP, _DW), lambda i: (i, 0)),
        pl.BlockSpec((_BP * _K // 128, 128), lambda i: (i, 0)),
        pl.BlockSpec((_BP, 8), lambda i: (i, 0)),
    ],
    out_shape=[
        jax.ShapeDtypeStruct((_B * _N, _DW), jnp.int32),
        jax.ShapeDtypeStruct((_B * _N * _K // 128, 128), jnp.int32),
        jax.ShapeDtypeStruct((_B * _N, 8), jnp.float32),
    ],
)


def _tc_body(x_ref, gx_ref, gm_ref, pq_ref, wst_ref, wext_ref, sm_ref, o_ref):
    xr = x_ref[0]
    acc = jnp.dot(xr, wst_ref[...], preferred_element_type=jnp.float32)
    acc = acc + jnp.dot(gx_ref[0], wext_ref[...],
                        preferred_element_type=jnp.float32)
    gm = gm_ref[0]
    pq = pq_ref[0]
    px, py, pz, q = pq[:, 0:1], pq[:, 1:2], pq[:, 2:3], pq[:, 3:4]
    mpx, mpz = gm[:, 0:1], gm[:, 1:2]
    mpy, mq = gm[:, 16:17], gm[:, 17:18]
    dterm = q - 2.0 * (px * mpx + py * mpy + pz * mpz) + mq
    sm = sm_ref[...]
    acc = acc + ((px - mpx) * sm[0:1, :] + (py - mpy) * sm[1:2, :]
                 + (pz - mpz) * sm[2:3, :] + dterm * sm[3:4, :])
    o_ref[0] = jnp.where(acc >= 0, acc, 0.2 * acc)


_BR = 1000

_tc_dense = pl.pallas_call(
    _tc_body,
    grid=(_B, _N // _BR),
    in_specs=[
        pl.BlockSpec((1, _BR, _C), lambda b, i: (b, i, 0)),
        pl.BlockSpec((1, _BR, _C), lambda b, i: (b, i, 0)),
        pl.BlockSpec((1, _BR, 32), lambda b, i: (b, i, 0)),
        pl.BlockSpec((1, _BR, 8), lambda b, i: (b, i, 0)),
        pl.BlockSpec((_C, _C), lambda b, i: (0, 0)),
        pl.BlockSpec((_C, _C), lambda b, i: (0, 0)),
        pl.BlockSpec((8, _C), lambda b, i: (0, 0)),
    ],
    out_specs=pl.BlockSpec((1, _BR, _C), lambda b, i: (b, i, 0)),
    out_shape=jax.ShapeDtypeStruct((_B, _N, _C), jnp.float32),
)


def kernel(x, pos, idx, W_self, W_edge):
    B, N, C = x.shape
    tab, idx2, pq = _tc_pack(
        x.reshape(B * N, C), pos.reshape(B * N, 3), idx.reshape(B * N, _K))
    gx, gm = _make_sc_gather_mean()(tab, idx2)
    gx = gx.reshape(B, _NP, C)
    gm = gm.reshape(B, _NP, 32)

    pq = pq.reshape(B, N, 8)
    wst = W_self.T
    wext = W_edge[:, :C].T
    sm = jnp.zeros((8, C), jnp.float32)
    sm = sm.at[0:3, :].set(W_edge[:, C:C + 3].T)
    sm = sm.at[3, :].set(W_edge[:, C + 3])
    return _tc_dense(x, gx, gm, pq, wst, wext, sm)


# speedup vs baseline: 1.0158x; 1.0158x over previous
import functools

import jax
import jax.numpy as jnp
from jax import lax
from jax.experimental import pallas as pl
from jax.experimental.pallas import tpu as pltpu
from jax.experimental.pallas import tpu_sc as plsc

_B, _N, _C, _K = 4, 10000, 128, 16
_DW = 80
_NC, _NS = 2, 16
_NP = 10240
_RP = _B * _NP
_CH = 8
_PER_T = _NP // _NS
_NCHUNK = _PER_T // _CH
_TROWS = _N // _NS
_HIMASK = -65536


@functools.cache
def _make_sc_gather_mean():
    mesh = plsc.VectorSubcoreMesh(
        core_axis_name="c", subcore_axis_name="s",
        num_cores=_NC, num_subcores=_NS)

    @functools.partial(
        pl.kernel,
        out_type=(
            jax.ShapeDtypeStruct((_RP, _C), jnp.float32),
            jax.ShapeDtypeStruct((_RP, 32), jnp.float32),
        ),
        mesh=mesh,
        compiler_params=pltpu.CompilerParams(
            use_tc_tiling_on_sc=False, needs_layout_passes=False),
        scratch_types=[
            pltpu.VMEM((_PER_T * _K // 128, 128), jnp.int32),
            pltpu.VMEM((4, _CH * _K, _DW), jnp.int32),
            pltpu.VMEM((2, _CH, _C), jnp.float32),
            pltpu.VMEM((2, _CH, 32), jnp.float32),
            pltpu.VMEM_SHARED((_N, _DW), jnp.int32),
            pltpu.SemaphoreType.DMA((4,)),
            pltpu.SemaphoreType.DMA((2,)),
            pltpu.SemaphoreType.DMA((2,)),
            pltpu.SemaphoreType.DMA,
        ],
    )
    def _sc_gather_mean(tab_hbm, idxg_hbm, outx_hbm, outm_hbm,
                        idx_v, rows_v, accx_v, accm_v, tab_sh,
                        rsem, sxsem, smsem, tsem):
        c = lax.axis_index("c")
        s = lax.axis_index("s")

        def _gather(j, sl):
            pltpu.async_copy(
                tab_sh.at[idx_v.at[j]],
                rows_v.at[sl], rsem.at[sl])

        def _gather_wait(sl):
            pltpu.make_async_copy(
                tab_sh.at[idx_v.at[0]],
                rows_v.at[sl], rsem.at[sl]).wait()

        inv_k = 1.0 / _K

        cbase = jnp.minimum(s * _PER_T, _N - _PER_T)

        for phase in range(2):
            b = c * 2 + phase
            pltpu.async_copy(
                tab_hbm.at[pl.ds(b * _N + s * _TROWS, _TROWS)],
                tab_sh.at[pl.ds(s * _TROWS, _TROWS)], tsem).wait()
            plsc.subcore_barrier()

            pltpu.sync_copy(
                idxg_hbm.at[pl.ds((b * _N + cbase) * _K // 128, _NCHUNK)],
                idx_v)
            base = b * _NP + cbase

            def _store(j, sl, base=base):
                row0 = base + j * _CH
                pltpu.async_copy(accx_v.at[sl],
                                 outx_hbm.at[pl.ds(row0, _CH)], sxsem.at[sl])
                pltpu.async_copy(accm_v.at[sl],
                                 outm_hbm.at[pl.ds(row0, _CH)], smsem.at[sl])

            def _store_wait(sl):
                pltpu.make_async_copy(
                    accx_v.at[sl], outx_hbm.at[pl.ds(0, _CH)],
                    sxsem.at[sl]).wait()
                pltpu.make_async_copy(
                    accm_v.at[sl], outm_hbm.at[pl.ds(0, _CH)],
                    smsem.at[sl]).wait()

            _gather(0, 0)
            _gather(1, 1)
            _gather(2, 2)

            @pl.loop(0, _NCHUNK, step=4)
            def _quad(j0):
                for sl in (0, 1, 2, 3):
                    j = j0 + sl
                    st = sl & 1

                    @pl.when(j + 3 < _NCHUNK)
                    def _():
                        _gather(j + 3, (sl + 3) & 3)

                    _gather_wait(sl)

                    @pl.when(j >= 2)
                    def _():
                        _store_wait(st)

                    @pl.loop(0, _CH)
                    def _center(cc):
                        r0 = cc * _K
                        for d in range(_DW // 16):
                            w0 = rows_v[sl, r0, pl.ds(d * 16, 16)]
                            alo = plsc.bitcast(w0 << 16, jnp.float32)
                            ahi = plsc.bitcast(w0, jnp.float32)
                            for k in range(1, _K):
                                wk = rows_v[sl, r0 + k, pl.ds(d * 16, 16)]
                                alo = alo + plsc.bitcast(wk << 16, jnp.float32)
                                ahi = ahi + plsc.bitcast(wk, jnp.float32)
                            alo = alo * inv_k
                            ahi = ahi * inv_k
                            if d < 4:
                                accx_v[st, cc, pl.ds(d * 16, 16)] = alo
                                accx_v[st, cc, pl.ds(64 + d * 16, 16)] = ahi
                            else:
                                accm_v[st, cc, pl.ds(0, 16)] = alo
                                accm_v[st, cc, pl.ds(16, 16)] = ahi

                    _store(j, st)

            _store_wait(0)
            _store_wait(1)
            plsc.subcore_barrier()

    return _sc_gather_mean


def _pack_body(x_ref, pos_ref, idx_ref, tab_ref, idx2_ref, pq_ref):
    him = jnp.uint32(0xffff0000)
    xb = lax.bitcast_convert_type(x_ref[...], jnp.uint32)
    wx = (xb[:, 64:128] & him) | (xb[:, 0:64] >> 16)
    p3 = pos_ref[...]
    px, py, pz = p3[:, 0:1], p3[:, 1:2], p3[:, 2:3]
    q = px * px + py * py + pz * pz
    pb = lax.bitcast_convert_type(
        jnp.concatenate([px, py, pz, q], axis=1), jnp.uint32)
    w64 = (pb[:, 1:2] & him) | (pb[:, 0:1] >> 16)
    w65 = (pb[:, 3:4] & him) | (pb[:, 2:3] >> 16)
    zpad = jnp.zeros((wx.shape[0], _DW - 66), jnp.uint32)
    tab_ref[...] = lax.bitcast_convert_type(
        jnp.concatenate([wx, w64, w65, zpad], axis=1), jnp.int32)
    pq_ref[...] = jnp.concatenate(
        [px, py, pz, q, jnp.zeros((wx.shape[0], 4), jnp.float32)], axis=1)
    ir3 = idx_ref[...].reshape(_BP // 8, 8, _K)
    for t in range(8):
        idx2_ref[:, pl.ds(_K * t, _K)] = ir3[:, t, :]


_BP = 1600

_tc_pack = pl.pallas_call(
    _pack_body,
    grid=(_B * _N // _BP,),
    in_specs=[
        pl.BlockSpec((_BP, _C), lambda i: (i, 0)),
        pl.BlockSpec((_BP, 3), lambda i: (i, 0)),
        pl.BlockSpec((_BP, _K), lambda i: (i, 0)),
    ],
    out_specs=[
        pl.BlockSpec((_BP, _DW), lambda i: (i, 0)),
        pl.BlockSpec((_BP * _K // 128, 128), lambda i: (i, 0)),
        pl.BlockSpec((_BP, 8), lambda i: (i, 0)),
    ],
    out_shape=[
        jax.ShapeDtypeStruct((_B * _N, _DW), jnp.int32),
        jax.ShapeDtypeStruct((_B * _N * _K // 128, 128), jnp.int32),
        jax.ShapeDtypeStruct((_B * _N, 8), jnp.float32),
    ],
)


def _tc_body(x_ref, gx_ref, gm_ref, pq_ref, wst_ref, wext_ref, sm_ref, o_ref):
    xr = x_ref[0]
    acc = jnp.dot(xr, wst_ref[...], preferred_element_type=jnp.float32)
    acc = acc + jnp.dot(gx_ref[0], wext_ref[...],
                        preferred_element_type=jnp.float32)
    gm = gm_ref[0]
    pq = pq_ref[0]
    px, py, pz, q = pq[:, 0:1], pq[:, 1:2], pq[:, 2:3], pq[:, 3:4]
    mpx, mpz = gm[:, 0:1], gm[:, 1:2]
    mpy, mq = gm[:, 16:17], gm[:, 17:18]
    dterm = q - 2.0 * (px * mpx + py * mpy + pz * mpz) + mq
    sm = sm_ref[...]
    acc = acc + ((px - mpx) * sm[0:1, :] + (py - mpy) * sm[1:2, :]
                 + (pz - mpz) * sm[2:3, :] + dterm * sm[3:4, :])
    o_ref[0] = jnp.where(acc >= 0, acc, 0.2 * acc)


_BR = 2000

_tc_dense = pl.pallas_call(
    _tc_body,
    grid=(_B, _N // _BR),
    in_specs=[
        pl.BlockSpec((1, _BR, _C), lambda b, i: (b, i, 0)),
        pl.BlockSpec((1, _BR, _C), lambda b, i: (b, i, 0)),
        pl.BlockSpec((1, _BR, 32), lambda b, i: (b, i, 0)),
        pl.BlockSpec((1, _BR, 8), lambda b, i: (b, i, 0)),
        pl.BlockSpec((_C, _C), lambda b, i: (0, 0)),
        pl.BlockSpec((_C, _C), lambda b, i: (0, 0)),
        pl.BlockSpec((8, _C), lambda b, i: (0, 0)),
    ],
    out_specs=pl.BlockSpec((1, _BR, _C), lambda b, i: (b, i, 0)),
    out_shape=jax.ShapeDtypeStruct((_B, _N, _C), jnp.float32),
)


def kernel(x, pos, idx, W_self, W_edge):
    B, N, C = x.shape
    tab, idx2, pq = _tc_pack(
        x.reshape(B * N, C), pos.reshape(B * N, 3), idx.reshape(B * N, _K))
    gx, gm = _make_sc_gather_mean()(tab, idx2)
    gx = gx.reshape(B, _NP, C)
    gm = gm.reshape(B, _NP, 32)

    pq = pq.reshape(B, N, 8)
    wst = W_self.T
    wext = W_edge[:, :C].T
    sm = jnp.zeros((8, C), jnp.float32)
    sm = sm.at[0:3, :].set(W_edge[:, C:C + 3].T)
    sm = sm.at[3, :].set(W_edge[:, C + 3])
    return _tc_dense(x, gx, gm, pq, wst, wext, sm)


# speedup vs baseline: 1.0176x; 1.0017x over previous
import functools

import jax
import jax.numpy as jnp
from jax import lax
from jax.experimental import pallas as pl
from jax.experimental.pallas import tpu as pltpu
from jax.experimental.pallas import tpu_sc as plsc

_B, _N, _C, _K = 4, 10000, 128, 16
_DW = 80
_NC, _NS = 2, 16
_NP = 10240
_RP = _B * _NP
_CH = 8
_PER_T = _NP // _NS
_NCHUNK = _PER_T // _CH
_TROWS = _N // _NS


@functools.cache
def _make_sc_gather_mean():
    mesh = plsc.VectorSubcoreMesh(
        core_axis_name="c", subcore_axis_name="s",
        num_cores=_NC, num_subcores=_NS)

    @functools.partial(
        pl.kernel,
        out_type=(
            jax.ShapeDtypeStruct((_RP, _C), jnp.float32),
            jax.ShapeDtypeStruct((_RP, 32), jnp.float32),
        ),
        mesh=mesh,
        compiler_params=pltpu.CompilerParams(
            use_tc_tiling_on_sc=False, needs_layout_passes=False),
        scratch_types=[
            pltpu.VMEM((_PER_T * _K // 128, 128), jnp.int32),
            pltpu.VMEM((4, _CH * _K, _DW), jnp.int32),
            pltpu.VMEM((2, _CH, _C), jnp.float32),
            pltpu.VMEM((2, _CH, 32), jnp.float32),
            pltpu.VMEM_SHARED((_N, _DW), jnp.int32),
            pltpu.SemaphoreType.DMA((4,)),
            pltpu.SemaphoreType.DMA((2,)),
            pltpu.SemaphoreType.DMA((2,)),
            pltpu.SemaphoreType.DMA,
        ],
    )
    def _sc_gather_mean(tab_hbm, idxg_hbm, outx_hbm, outm_hbm,
                        idx_v, rows_v, accx_v, accm_v, tab_sh,
                        rsem, sxsem, smsem, tsem):
        c = lax.axis_index("c")
        s = lax.axis_index("s")

        def _gather(j, sl):
            pltpu.async_copy(
                tab_sh.at[idx_v.at[j]],
                rows_v.at[sl], rsem.at[sl])

        def _gather_wait(sl):
            pltpu.make_async_copy(
                tab_sh.at[idx_v.at[0]],
                rows_v.at[sl], rsem.at[sl]).wait()

        inv_k = 1.0 / _K

        cbase = jnp.minimum(s * _PER_T, _N - _PER_T)

        for phase in range(2):
            b = c * 2 + phase
            pltpu.async_copy(
                tab_hbm.at[pl.ds(b * _N + s * _TROWS, _TROWS)],
                tab_sh.at[pl.ds(s * _TROWS, _TROWS)], tsem).wait()
            plsc.subcore_barrier()

            pltpu.sync_copy(
                idxg_hbm.at[pl.ds((b * _N + cbase) * _K // 128, _NCHUNK)],
                idx_v)
            base = b * _NP + cbase

            def _store(j, sl, base=base):
                row0 = base + j * _CH
                pltpu.async_copy(accx_v.at[sl],
                                 outx_hbm.at[pl.ds(row0, _CH)], sxsem.at[sl])
                pltpu.async_copy(accm_v.at[sl],
                                 outm_hbm.at[pl.ds(row0, _CH)], smsem.at[sl])

            def _store_wait(sl):
                pltpu.make_async_copy(
                    accx_v.at[sl], outx_hbm.at[pl.ds(0, _CH)],
                    sxsem.at[sl]).wait()
                pltpu.make_async_copy(
                    accm_v.at[sl], outm_hbm.at[pl.ds(0, _CH)],
                    smsem.at[sl]).wait()

            _gather(0, 0)
            _gather(1, 1)
            _gather(2, 2)

            @pl.loop(0, _NCHUNK, step=4)
            def _quad(j0):
                for sl in (0, 1, 2, 3):
                    j = j0 + sl
                    st = sl & 1

                    @pl.when(j + 3 < _NCHUNK)
                    def _():
                        _gather(j + 3, (sl + 3) & 3)

                    _gather_wait(sl)

                    @pl.when(j >= 2)
                    def _():
                        _store_wait(st)

                    @pl.loop(0, _CH)
                    def _center(cc):
                        r0 = cc * _K
                        for d in range(_DW // 16):
                            w0 = rows_v[sl, r0, pl.ds(d * 16, 16)]
                            alo = plsc.bitcast(w0 << 16, jnp.float32)
                            ahi = plsc.bitcast(w0, jnp.float32)
                            for k in range(1, _K):
                                wk = rows_v[sl, r0 + k, pl.ds(d * 16, 16)]
                                alo = alo + plsc.bitcast(wk << 16, jnp.float32)
                                ahi = ahi + plsc.bitcast(wk, jnp.float32)
                            alo = alo * inv_k
                            ahi = ahi * inv_k
                            if d < 4:
                                accx_v[st, cc, pl.ds(d * 16, 16)] = alo
                                accx_v[st, cc, pl.ds(64 + d * 16, 16)] = ahi
                            else:
                                accm_v[st, cc, pl.ds(0, 16)] = alo
                                accm_v[st, cc, pl.ds(16, 16)] = ahi

                    _store(j, st)

            _store_wait(0)
            _store_wait(1)
            plsc.subcore_barrier()

    return _sc_gather_mean


def _pack_body(x_ref, pos_ref, idx_ref, tab_ref, idx2_ref, pq_ref):
    him = jnp.uint32(0xffff0000)
    xb = lax.bitcast_convert_type(x_ref[...], jnp.uint32)
    wx = (xb[:, 64:128] & him) | (xb[:, 0:64] >> 16)
    p3 = pos_ref[...]
    px, py, pz = p3[:, 0:1], p3[:, 1:2], p3[:, 2:3]
    q = px * px + py * py + pz * pz
    pb = lax.bitcast_convert_type(
        jnp.concatenate([px, py, pz, q], axis=1), jnp.uint32)
    w64 = (pb[:, 1:2] & him) | (pb[:, 0:1] >> 16)
    w65 = (pb[:, 3:4] & him) | (pb[:, 2:3] >> 16)
    zpad = jnp.zeros((wx.shape[0], _DW - 66), jnp.uint32)
    tab_ref[...] = lax.bitcast_convert_type(
        jnp.concatenate([wx, w64, w65, zpad], axis=1), jnp.int32)
    pq_ref[...] = jnp.concatenate(
        [px, py, pz, q, jnp.zeros((wx.shape[0], 4), jnp.float32)], axis=1)
    ir3 = idx_ref[...].reshape(_BP // 8, 8, _K)
    for t in range(8):
        idx2_ref[:, pl.ds(_K * t, _K)] = ir3[:, t, :]


_BP = 1600

_tc_pack = pl.pallas_call(
    _pack_body,
    grid=(_B * _N // _BP,),
    in_specs=[
        pl.BlockSpec((_BP, _C), lambda i: (i, 0)),
        pl.BlockSpec((_BP, 3), lambda i: (i, 0)),
        pl.BlockSpec((_BP, _K), lambda i: (i, 0)),
    ],
    out_specs=[
        pl.BlockSpec((_BP, _DW), lambda i: (i, 0)),
        pl.BlockSpec((_BP * _K // 128, 128), lambda i: (i, 0)),
        pl.BlockSpec((_BP, 8), lambda i: (i, 0)),
    ],
    out_shape=[
        jax.ShapeDtypeStruct((_B * _N, _DW), jnp.int32),
        jax.ShapeDtypeStruct((_B * _N * _K // 128, 128), jnp.int32),
        jax.ShapeDtypeStruct((_B * _N, 8), jnp.float32),
    ],
)


def _tc_body(x_ref, gx_ref, gm_ref, pq_ref, wst_ref, wext_ref, sm_ref, o_ref):
    xr = x_ref[0]
    acc = jnp.dot(xr, wst_ref[...], preferred_element_type=jnp.float32)
    acc = acc + jnp.dot(gx_ref[0], wext_ref[...],
                        preferred_element_type=jnp.float32)
    gm = gm_ref[0]
    pq = pq_ref[0]
    px, py, pz, q = pq[:, 0:1], pq[:, 1:2], pq[:, 2:3], pq[:, 3:4]
    mpx, mpz = gm[:, 0:1], gm[:, 1:2]
    mpy, mq = gm[:, 16:17], gm[:, 17:18]
    dterm = q - 2.0 * (px * mpx + py * mpy + pz * mpz) + mq
    sm = sm_ref[...]
    acc = acc + ((px - mpx) * sm[0:1, :] + (py - mpy) * sm[1:2, :]
                 + (pz - mpz) * sm[2:3, :] + dterm * sm[3:4, :])
    o_ref[0] = jnp.where(acc >= 0, acc, 0.2 * acc)


_BR = 2000

_tc_dense = pl.pallas_call(
    _tc_body,
    grid=(_B, _N // _BR),
    in_specs=[
        pl.BlockSpec((1, _BR, _C), lambda b, i: (b, i, 0)),
        pl.BlockSpec((1, _BR, _C), lambda b, i: (b, i, 0)),
        pl.BlockSpec((1, _BR, 32), lambda b, i: (b, i, 0)),
        pl.BlockSpec((1, _BR, 8), lambda b, i: (b, i, 0)),
        pl.BlockSpec((_C, _C), lambda b, i: (0, 0)),
        pl.BlockSpec((_C, _C), lambda b, i: (0, 0)),
        pl.BlockSpec((8, _C), lambda b, i: (0, 0)),
    ],
    out_specs=pl.BlockSpec((1, _BR, _C), lambda b, i: (b, i, 0)),
    out_shape=jax.ShapeDtypeStruct((_B, _N, _C), jnp.float32),
)


def kernel(x, pos, idx, W_self, W_edge):
    B, N, C = x.shape
    tab, idx2, pq = _tc_pack(
        x.reshape(B * N, C), pos.reshape(B * N, 3), idx.reshape(B * N, _K))
    gx, gm = _make_sc_gather_mean()(tab, idx2)
    gx = gx.reshape(B, _NP, C)
    gm = gm.reshape(B, _NP, 32)

    pq = pq.reshape(B, N, 8)
    wst = W_self.T
    wext = W_edge[:, :C].T
    sm = jnp.zeros((8, C), jnp.float32)
    sm = sm.at[0:3, :].set(W_edge[:, C:C + 3].T)
    sm = sm.at[3, :].set(W_edge[:, C + 3])
    return _tc_dense(x, gx, gm, pq, wst, wext, sm)
